# baseline (speedup 1.0000x reference)
import functools

import jax
import jax.numpy as jnp
from jax import lax
from jax.experimental import pallas as pl
from jax.experimental.pallas import tpu as pltpu
from jax.experimental.pallas import tpu_sc as plsc

_L = 200
_E = 128
_NW = 32
_C = 128
_NS = 4


def _build(batch):
  total = batch * _L
  cpw = total // (_NW * _C)
  assert cpw * _NW * _C == total and cpw % _NS == 0

  mesh = plsc.VectorSubcoreMesh(core_axis_name="c", subcore_axis_name="s")

  @functools.partial(
      pl.kernel,
      mesh=mesh,
      out_type=jax.ShapeDtypeStruct((total, _E), jnp.float32),
      scratch_types=[
          pltpu.VMEM((cpw, _C), jnp.int32),
          pltpu.VMEM((_NS, _C, _E), jnp.float32),
          pltpu.VMEM((_L, _E), jnp.float32),
          [pltpu.SemaphoreType.DMA] * _NS,
          [pltpu.SemaphoreType.DMA] * _NS,
      ],
  )
  def k(idx_hbm, text_hbm, pos_hbm, out_hbm, idx_v, buf_v, pos_v, gsem, ssem):
    wid = lax.axis_index("s") * 2 + lax.axis_index("c")
    base = wid * cpw

    pltpu.sync_copy(pos_hbm, pos_v)
    pltpu.sync_copy(idx_hbm.at[pl.ds(base, cpw)], idx_v)

    def gather_start(cl, s):
      pltpu.async_copy(text_hbm.at[idx_v.at[cl]], buf_v.at[s], gsem[s])

    def gather_wait(cl, s):
      pltpu.make_async_copy(text_hbm.at[idx_v.at[cl]], buf_v.at[s],
                            gsem[s]).wait()

    def store_start(cl, s):
      pltpu.async_copy(buf_v.at[s], out_hbm.at[pl.ds((base + cl) * _C, _C)],
                       ssem[s])

    def store_wait(cl, s):
      pltpu.make_async_copy(buf_v.at[s],
                            out_hbm.at[pl.ds((base + cl) * _C, _C)],
                            ssem[s]).wait()

    def add_pos(cl, s):
      off = lax.rem((base + cl) * _C, _L)

      @pl.loop(0, _C, unroll=2)
      def _(r):
        p = off + r
        p = jnp.where(p >= _L, p - _L, p)
        for j in range(_E // 16):
          sl = pl.ds(j * 16, 16)
          plsc.addupdate(buf_v.at[s, r, sl], pos_v[p, sl])

    gather_start(0, 0)

    @pl.loop(0, cpw, step=_NS)
    def _(c0):
      for b in range(_NS):
        cl = c0 + b
        ns = (b + 1) % _NS

        @pl.when(jnp.logical_and(cl + 1 < cpw, cl >= _NS - 1))
        def _():
          store_wait(cl - (_NS - 1), ns)

        @pl.when(cl + 1 < cpw)
        def _():
          gather_start(cl + 1, ns)

        gather_wait(cl, b)
        add_pos(cl, b)
        store_start(cl, b)

    for b in range(_NS):
      store_wait(cpw - _NS + b, b)

  return k


def kernel(inputs, text_table, pos_table):
  batch, seq = inputs.shape
  assert seq == _L and text_table.shape[1] == _E
  idx2d = inputs.reshape(batch * _L // _C, _C).astype(jnp.int32)
  out = _build(batch)(idx2d, text_table, pos_table)
  return out.reshape(batch, _L, _E)


# speedup vs baseline: 2.2912x; 2.2912x over previous
import functools

import jax
import jax.numpy as jnp
from jax import lax
from jax.experimental import pallas as pl
from jax.experimental.pallas import tpu as pltpu
from jax.experimental.pallas import tpu_sc as plsc

_L = 200
_E = 128
_NW = 32
_H = _L // 2
_NS = 3


def _maybe(cond, fn):
  if isinstance(cond, (bool, int)):
    if cond:
      fn()
  else:
    pl.when(cond)(fn)


def _build(batch):
  total = batch * _L
  cpw = total // (_NW * _L)
  assert cpw * _NW * _L == total and cpw > 2 * _NS

  mesh = plsc.VectorSubcoreMesh(core_axis_name="c", subcore_axis_name="s")

  @functools.partial(
      pl.kernel,
      mesh=mesh,
      out_type=jax.ShapeDtypeStruct((total, _E), jnp.float32),
      scratch_types=[
          pltpu.VMEM((_NS, 2, _H), jnp.int32),
          pltpu.VMEM((_NS, _L, _E), jnp.float32),
          pltpu.VMEM((_L, _E), jnp.float32),
          [pltpu.SemaphoreType.DMA] * _NS,
          [pltpu.SemaphoreType.DMA] * _NS,
          [pltpu.SemaphoreType.DMA] * _NS,
      ],
  )
  def k(idx_hbm, text_hbm, pos_hbm, out_hbm, idx_v, buf_v, pos_v,
        isem, gsem, ssem):
    wid = lax.axis_index("s") * 2 + lax.axis_index("c")
    base = wid * cpw

    pltpu.sync_copy(pos_hbm, pos_v)

    def idx_start(cl, s):
      pltpu.async_copy(idx_hbm.at[pl.ds(2 * (base + cl), 2)], idx_v.at[s],
                       isem[s])

    def idx_wait(cl, s):
      pltpu.make_async_copy(idx_hbm.at[pl.ds(2 * (base + cl), 2)],
                            idx_v.at[s], isem[s]).wait()

    def gather_start(cl, s):
      pltpu.async_copy(text_hbm.at[idx_v.at[s, 0]],
                       buf_v.at[s, pl.ds(0, _H)], gsem[s])
      pltpu.async_copy(text_hbm.at[idx_v.at[s, 1]],
                       buf_v.at[s, pl.ds(_H, _H)], gsem[s])

    def gather_wait(cl, s):
      pltpu.make_async_copy(text_hbm.at[idx_v.at[s, 0]],
                            buf_v.at[s, pl.ds(0, _H)], gsem[s]).wait()
      pltpu.make_async_copy(text_hbm.at[idx_v.at[s, 1]],
                            buf_v.at[s, pl.ds(_H, _H)], gsem[s]).wait()

    def store_start(cl, s):
      pltpu.async_copy(buf_v.at[s], out_hbm.at[pl.ds((base + cl) * _L, _L)],
                       ssem[s])

    def store_wait(cl, s):
      pltpu.make_async_copy(buf_v.at[s],
                            out_hbm.at[pl.ds((base + cl) * _L, _L)],
                            ssem[s]).wait()

    def add_pos(s):
      @pl.loop(0, _L, unroll=2)
      def _(r):
        for j in range(_E // 16):
          sl = pl.ds(j * 16, 16)
          plsc.addupdate(buf_v.at[s, r, sl], pos_v[r, sl])

    def step(cl, s):
      ns = (s + 1) % _NS
      nns = (s + 2) % _NS

      _maybe(cl + 2 < cpw, lambda: idx_start(cl + 2, nns))

      def _next():
        _maybe(cl >= _NS - 1, lambda: store_wait(cl - (_NS - 1), ns))
        idx_wait(cl + 1, ns)
        gather_start(cl + 1, ns)

      _maybe(cl + 1 < cpw, _next)

      gather_wait(cl, s)
      add_pos(s)
      store_start(cl, s)

    idx_start(0, 0)
    idx_wait(0, 0)
    gather_start(0, 0)
    idx_start(1, 1)

    body = cpw - cpw % _NS

    @pl.loop(0, body, step=_NS)
    def _(c0):
      for b in range(_NS):
        step(c0 + b, b)

    for cl in range(body, cpw):
      step(cl, cl % _NS)

    for cl in range(cpw - _NS, cpw):
      store_wait(cl, cl % _NS)

  return k


def kernel(inputs, text_table, pos_table):
  batch, seq = inputs.shape
  assert seq == _L and text_table.shape[1] == _E
  idx2d = inputs.reshape(batch * _L // _H, _H).astype(jnp.int32)
  out = _build(batch)(idx2d, text_table, pos_table)
  return out.reshape(batch, _L, _E)
